# baseline (speedup 1.0000x reference)
import functools

import jax
import jax.numpy as jnp
from jax import lax
from jax.experimental import pallas as pl
from jax.experimental.pallas import tpu as pltpu
from jax.experimental.pallas import tpu_sc as plsc

_C = 32
_TT = 128
_KC = 128


def _tc_body(off_ref, xs_ref, w_ref, b_ref, out_ref, acc_ref):
    g = pl.program_id(0)
    ncols = out_ref.shape[1]

    @pl.when(g == 0)
    def _init():
        out_ref[...] = jnp.zeros(out_ref.shape, jnp.float32)

    rs = off_ref[g * _C]
    re = off_ref[g * _C + _C]
    base0 = (rs // 8) * 8
    nblk = (re - base0 + _TT - 1) // _TT

    def blk(k, carry):
        base = base0 + k * _TT
        xblk = xs_ref[pl.ds(base, _TT), :]
        riota = base + lax.broadcasted_iota(jnp.int32, (_TT, 1), 0)
        acc = jnp.zeros((_TT, ncols), jnp.float32)
        for j in range(_C):
            oe = off_ref[g * _C + j]
            oe1 = off_ref[g * _C + j + 1]
            m = (riota >= oe) & (riota < oe1)
            dotj = jnp.dot(xblk, w_ref[j], preferred_element_type=jnp.float32)
            brow = b_ref[pl.ds(j, 1), :]
            acc = acc + jnp.where(m, dotj + brow, 0.0)
        out_ref[pl.ds(base, _TT), :] = out_ref[pl.ds(base, _TT), :] + acc
        return carry

    lax.fori_loop(0, nblk, blk, 0)


def _grouped_matmul(off, xs_pad, W, b, interpret=False):
    E, D, _ = W.shape
    B_pad = xs_pad.shape[0]
    grid = (E // _C,)
    grid_spec = pltpu.PrefetchScalarGridSpec(
        num_scalar_prefetch=1,
        grid=grid,
        in_specs=[
            pl.BlockSpec((B_pad, D), lambda g, off_ref: (0, 0)),
            pl.BlockSpec((_C, D, D), lambda g, off_ref: (g, 0, 0)),
            pl.BlockSpec((_C, D), lambda g, off_ref: (g, 0)),
        ],
        out_specs=pl.BlockSpec((B_pad, D), lambda g, off_ref: (0, 0)),
        scratch_shapes=[pltpu.VMEM((_TT, D), jnp.float32)],
    )
    return pl.pallas_call(
        _tc_body,
        grid_spec=grid_spec,
        out_shape=jax.ShapeDtypeStruct((B_pad, D), jnp.float32),
        interpret=interpret,
    )(off, xs_pad, W, b)


def _sc_mesh():
    return plsc.VectorSubcoreMesh(core_axis_name="c", subcore_axis_name="s")


def _sc_hist_rank(ids, E, NW):
    B = ids.shape[0]
    tpw = B // NW
    TPL = tpw // 16

    @functools.partial(
        pl.kernel,
        mesh=_sc_mesh(),
        out_type=(
            jax.ShapeDtypeStruct((NW, E * 16), jnp.int32),
            jax.ShapeDtypeStruct((B,), jnp.int32),
        ),
        scratch_types=[
            pltpu.VMEM((tpw,), jnp.int32),
            pltpu.VMEM((E * 16,), jnp.int32),
            pltpu.VMEM((tpw,), jnp.int32),
        ],
        compiler_params=pltpu.CompilerParams(needs_layout_passes=False),
    )
    def k(ids_hbm, hist_hbm, rank_hbm, ids_v, hist_v, rank_v):
        wid = lax.axis_index("s") * 2 + lax.axis_index("c")
        base = wid * tpw
        lane = lax.iota(jnp.int32, 16)
        pltpu.sync_copy(ids_hbm.at[pl.ds(base, tpw)], ids_v)

        def zero(i, _):
            for u in range(16):
                hist_v[pl.ds(i * 256 + u * 16, 16)] = jnp.zeros((16,), jnp.int32)
            return 0

        lax.fori_loop(0, E // 16, zero, 0)

        def body(j, _):
            idx = lane * TPL + j
            idvec = plsc.load_gather(ids_v, [idx])
            hidx = idvec * 16 + lane
            c = plsc.load_gather(hist_v, [hidx])
            plsc.store_scatter(hist_v, [hidx], c + 1)
            plsc.store_scatter(rank_v, [idx], c)
            return 0

        lax.fori_loop(0, TPL, body, 0)
        pltpu.sync_copy(hist_v, hist_hbm.at[wid])
        pltpu.sync_copy(rank_v, rank_hbm.at[pl.ds(base, tpw)])

    return k(ids)


def _sc_scatter_x(x, ids, rank, bw_all, B_pad, NW):
    B, D = x.shape
    tpw = B // NW
    TPL = tpw // 16
    nchunk = tpw // _KC
    E16 = bw_all.shape[1]

    @functools.partial(
        pl.kernel,
        mesh=_sc_mesh(),
        out_type=(
            jax.ShapeDtypeStruct((B_pad, D), jnp.float32),
            jax.ShapeDtypeStruct((NW, nchunk, _KC), jnp.int32),
        ),
        scratch_types=[
            pltpu.VMEM((tpw,), jnp.int32),
            pltpu.VMEM((tpw,), jnp.int32),
            pltpu.VMEM((E16,), jnp.int32),
            pltpu.VMEM((nchunk, _KC), jnp.int32),
            pltpu.VMEM((tpw, D), jnp.float32),
            pltpu.SemaphoreType.DMA,
        ],
        compiler_params=pltpu.CompilerParams(needs_layout_passes=False),
    )
    def k(x_hbm, ids_hbm, rank_hbm, bw_hbm, xs_hbm, pos_hbm,
          ids_v, rank_v, bw_v, pos_v, rows_v, sem):
        wid = lax.axis_index("s") * 2 + lax.axis_index("c")
        base = wid * tpw
        pltpu.sync_copy(ids_hbm.at[pl.ds(base, tpw)], ids_v)
        pltpu.sync_copy(rank_hbm.at[pl.ds(base, tpw)], rank_v)
        pltpu.sync_copy(bw_hbm.at[wid], bw_v)
        pltpu.sync_copy(x_hbm.at[pl.ds(base, tpw)], rows_v)
        for g in range(tpw // 16):
            sublane = (g * 16) // TPL
            sl = pl.ds(g * 16, 16)
            idvec = ids_v[sl]
            rvec = rank_v[sl]
            bvec = plsc.load_gather(bw_v, [idvec * 16 + sublane])
            pos_v[g // (_KC // 16), pl.ds((g % (_KC // 16)) * 16, 16)] = (
                bvec + rvec
            )
        copies = [
            pltpu.async_copy(
                rows_v.at[pl.ds(j * _KC, _KC)],
                xs_hbm.at[pos_v.at[j]],
                sem,
            )
            for j in range(nchunk)
        ]
        for c in copies:
            c.wait()
        pltpu.sync_copy(pos_v, pos_hbm.at[wid])

    return k(x, ids, rank, bw_all)


def _sc_gather_out(out_s, pos3, B, NW):
    D = out_s.shape[1]
    nchunk = pos3.shape[1]
    tpw = nchunk * _KC

    @functools.partial(
        pl.kernel,
        mesh=_sc_mesh(),
        out_type=jax.ShapeDtypeStruct((B, D), jnp.float32),
        scratch_types=[
            pltpu.VMEM((nchunk, _KC), jnp.int32),
            pltpu.VMEM((tpw, D), jnp.float32),
            pltpu.SemaphoreType.DMA,
        ],
    )
    def k(src_hbm, pos_hbm, out_hbm, idx_v, rows_v, sem):
        wid = lax.axis_index("s") * 2 + lax.axis_index("c")
        base = wid * tpw
        pltpu.sync_copy(pos_hbm.at[wid], idx_v)
        copies = [
            pltpu.async_copy(
                src_hbm.at[idx_v.at[j]],
                rows_v.at[pl.ds(j * _KC, _KC)],
                sem,
            )
            for j in range(nchunk)
        ]
        for c in copies:
            c.wait()
        pltpu.sync_copy(rows_v, out_hbm.at[pl.ds(base, tpw)])

    return k(out_s, pos3)


def kernel(x, ids, W, b, interpret=False):
    B, D = x.shape
    E = W.shape[0]
    B_pad = B + _TT
    NW = 32
    ids32 = ids.astype(jnp.int32)

    if interpret:
        tok = lax.iota(jnp.int32, B)
        sorted_ids, order = lax.sort((ids32, tok), num_keys=1)
        off = jnp.searchsorted(
            sorted_ids, lax.iota(jnp.int32, E + 1), side="left"
        ).astype(jnp.int32)
        xs_pad = jnp.pad(jnp.take(x, order, axis=0), ((0, _TT), (0, 0)))
        out_s = _grouped_matmul(off, xs_pad, W, b, interpret=True)
        return jnp.zeros((B, D), jnp.float32).at[order].set(out_s[:B])

    hist, rank = _sc_hist_rank(ids32, E, NW)
    hsw = hist.reshape(NW, E, 16).transpose(0, 2, 1).reshape(NW * 16, E)
    colsum = hsw.sum(axis=0)
    off = jnp.concatenate(
        [jnp.zeros((1,), jnp.int32), jnp.cumsum(colsum, dtype=jnp.int32)]
    )
    basew = (
        off[:E][None, :]
        + jnp.cumsum(hsw, axis=0, dtype=jnp.int32)
        - hsw
    )
    bw_all = basew.reshape(NW, 16, E).transpose(0, 2, 1).reshape(NW, E * 16)

    xs_pad, pos3 = _sc_scatter_x(x, ids32, rank, bw_all, B_pad, NW)
    out_s = _grouped_matmul(off, xs_pad, W, b)
    out = _sc_gather_out(out_s, pos3, B, NW)
    return out


# speedup vs baseline: 1.1077x; 1.1077x over previous
import functools

import jax
import jax.numpy as jnp
from jax import lax
from jax.experimental import pallas as pl
from jax.experimental.pallas import tpu as pltpu
from jax.experimental.pallas import tpu_sc as plsc

_C = 16
_TT = 256
_KC = 128


def _tc_body(off_ref, xs_ref, w_ref, b_ref, out_ref, acc_ref):
    g = pl.program_id(0)
    ncols = out_ref.shape[1]

    @pl.when(g == 0)
    def _init():
        out_ref[...] = jnp.zeros(out_ref.shape, jnp.float32)

    rs = off_ref[g * _C]
    re = off_ref[g * _C + _C]
    base0 = (rs // 8) * 8
    nblk = (re - base0 + _TT - 1) // _TT

    def blk(k, carry):
        base = base0 + k * _TT
        xblk = xs_ref[pl.ds(base, _TT), :]
        riota = base + lax.broadcasted_iota(jnp.int32, (_TT, 1), 0)
        acc = jnp.zeros((_TT, ncols), jnp.float32)
        for j in range(_C):
            oe = off_ref[g * _C + j]
            oe1 = off_ref[g * _C + j + 1]
            m = (riota >= oe) & (riota < oe1)
            dotj = jnp.dot(xblk, w_ref[j], preferred_element_type=jnp.float32)
            brow = b_ref[pl.ds(j, 1), :]
            acc = acc + jnp.where(m, dotj + brow, 0.0)
        out_ref[pl.ds(base, _TT), :] = out_ref[pl.ds(base, _TT), :] + acc
        return carry

    lax.fori_loop(0, nblk, blk, 0)


def _grouped_matmul(off, xs_pad, W, b, interpret=False):
    E, D, _ = W.shape
    B_pad = xs_pad.shape[0]
    grid = (E // _C,)
    grid_spec = pltpu.PrefetchScalarGridSpec(
        num_scalar_prefetch=1,
        grid=grid,
        in_specs=[
            pl.BlockSpec((B_pad, D), lambda g, off_ref: (0, 0)),
            pl.BlockSpec((_C, D, D), lambda g, off_ref: (g, 0, 0)),
            pl.BlockSpec((_C, D), lambda g, off_ref: (g, 0)),
        ],
        out_specs=pl.BlockSpec((B_pad, D), lambda g, off_ref: (0, 0)),
        scratch_shapes=[pltpu.VMEM((_TT, D), jnp.float32)],
    )
    return pl.pallas_call(
        _tc_body,
        grid_spec=grid_spec,
        out_shape=jax.ShapeDtypeStruct((B_pad, D), jnp.float32),
        interpret=interpret,
    )(off, xs_pad, W, b)


def _sc_mesh():
    return plsc.VectorSubcoreMesh(core_axis_name="c", subcore_axis_name="s")


def _sc_hist_rank(ids, E, NW):
    B = ids.shape[0]
    tpw = B // NW
    TPL = tpw // 16

    @functools.partial(
        pl.kernel,
        mesh=_sc_mesh(),
        out_type=(
            jax.ShapeDtypeStruct((NW, E * 16), jnp.int32),
            jax.ShapeDtypeStruct((B,), jnp.int32),
        ),
        scratch_types=[
            pltpu.VMEM((tpw,), jnp.int32),
            pltpu.VMEM((E * 16,), jnp.int32),
            pltpu.VMEM((tpw,), jnp.int32),
        ],
        compiler_params=pltpu.CompilerParams(needs_layout_passes=False),
    )
    def k(ids_hbm, hist_hbm, rank_hbm, ids_v, hist_v, rank_v):
        wid = lax.axis_index("s") * 2 + lax.axis_index("c")
        base = wid * tpw
        lane = lax.iota(jnp.int32, 16)
        pltpu.sync_copy(ids_hbm.at[pl.ds(base, tpw)], ids_v)

        def zero(i, _):
            for u in range(16):
                hist_v[pl.ds(i * 256 + u * 16, 16)] = jnp.zeros((16,), jnp.int32)
            return 0

        lax.fori_loop(0, E // 16, zero, 0)

        def body(j, _):
            idx = lane * TPL + j
            idvec = plsc.load_gather(ids_v, [idx])
            hidx = idvec * 16 + lane
            c = plsc.load_gather(hist_v, [hidx])
            plsc.store_scatter(hist_v, [hidx], c + 1)
            plsc.store_scatter(rank_v, [idx], c)
            return 0

        lax.fori_loop(0, TPL, body, 0)
        pltpu.sync_copy(hist_v, hist_hbm.at[wid])
        pltpu.sync_copy(rank_v, rank_hbm.at[pl.ds(base, tpw)])

    return k(ids)


def _sc_scatter_x(x, ids, rank, bw_all, B_pad, NW):
    B, D = x.shape
    tpw = B // NW
    TPL = tpw // 16
    nchunk = tpw // _KC
    E16 = bw_all.shape[1]

    @functools.partial(
        pl.kernel,
        mesh=_sc_mesh(),
        out_type=(
            jax.ShapeDtypeStruct((B_pad, D), jnp.float32),
            jax.ShapeDtypeStruct((NW, nchunk, _KC), jnp.int32),
        ),
        scratch_types=[
            pltpu.VMEM((tpw,), jnp.int32),
            pltpu.VMEM((tpw,), jnp.int32),
            pltpu.VMEM((E16,), jnp.int32),
            pltpu.VMEM((nchunk, _KC), jnp.int32),
            pltpu.VMEM((tpw, D), jnp.float32),
            pltpu.SemaphoreType.DMA,
        ],
        compiler_params=pltpu.CompilerParams(needs_layout_passes=False),
    )
    def k(x_hbm, ids_hbm, rank_hbm, bw_hbm, xs_hbm, pos_hbm,
          ids_v, rank_v, bw_v, pos_v, rows_v, sem):
        wid = lax.axis_index("s") * 2 + lax.axis_index("c")
        base = wid * tpw
        pltpu.sync_copy(ids_hbm.at[pl.ds(base, tpw)], ids_v)
        pltpu.sync_copy(rank_hbm.at[pl.ds(base, tpw)], rank_v)
        pltpu.sync_copy(bw_hbm.at[wid], bw_v)
        pltpu.sync_copy(x_hbm.at[pl.ds(base, tpw)], rows_v)
        for g in range(tpw // 16):
            sublane = (g * 16) // TPL
            sl = pl.ds(g * 16, 16)
            idvec = ids_v[sl]
            rvec = rank_v[sl]
            bvec = plsc.load_gather(bw_v, [idvec * 16 + sublane])
            pos_v[g // (_KC // 16), pl.ds((g % (_KC // 16)) * 16, 16)] = (
                bvec + rvec
            )
        copies = [
            pltpu.async_copy(
                rows_v.at[pl.ds(j * _KC, _KC)],
                xs_hbm.at[pos_v.at[j]],
                sem,
            )
            for j in range(nchunk)
        ]
        for c in copies:
            c.wait()
        pltpu.sync_copy(pos_v, pos_hbm.at[wid])

    return k(x, ids, rank, bw_all)


def _sc_gather_out(out_s, pos3, B, NW):
    D = out_s.shape[1]
    nchunk = pos3.shape[1]
    tpw = nchunk * _KC

    @functools.partial(
        pl.kernel,
        mesh=_sc_mesh(),
        out_type=jax.ShapeDtypeStruct((B, D), jnp.float32),
        scratch_types=[
            pltpu.VMEM((nchunk, _KC), jnp.int32),
            pltpu.VMEM((tpw, D), jnp.float32),
            pltpu.SemaphoreType.DMA,
        ],
    )
    def k(src_hbm, pos_hbm, out_hbm, idx_v, rows_v, sem):
        wid = lax.axis_index("s") * 2 + lax.axis_index("c")
        base = wid * tpw
        pltpu.sync_copy(pos_hbm.at[wid], idx_v)
        copies = [
            pltpu.async_copy(
                src_hbm.at[idx_v.at[j]],
                rows_v.at[pl.ds(j * _KC, _KC)],
                sem,
            )
            for j in range(nchunk)
        ]
        for c in copies:
            c.wait()
        pltpu.sync_copy(rows_v, out_hbm.at[pl.ds(base, tpw)])

    return k(out_s, pos3)


def kernel(x, ids, W, b, interpret=False):
    B, D = x.shape
    E = W.shape[0]
    B_pad = B + _TT
    NW = 32
    ids32 = ids.astype(jnp.int32)

    if interpret:
        tok = lax.iota(jnp.int32, B)
        sorted_ids, order = lax.sort((ids32, tok), num_keys=1)
        off = jnp.searchsorted(
            sorted_ids, lax.iota(jnp.int32, E + 1), side="left"
        ).astype(jnp.int32)
        xs_pad = jnp.pad(jnp.take(x, order, axis=0), ((0, _TT), (0, 0)))
        out_s = _grouped_matmul(off, xs_pad, W, b, interpret=True)
        return jnp.zeros((B, D), jnp.float32).at[order].set(out_s[:B])

    hist, rank = _sc_hist_rank(ids32, E, NW)
    hsw = hist.reshape(NW, E, 16).transpose(0, 2, 1).reshape(NW * 16, E)
    colsum = hsw.sum(axis=0)
    off = jnp.concatenate(
        [jnp.zeros((1,), jnp.int32), jnp.cumsum(colsum, dtype=jnp.int32)]
    )
    basew = (
        off[:E][None, :]
        + jnp.cumsum(hsw, axis=0, dtype=jnp.int32)
        - hsw
    )
    bw_all = basew.reshape(NW, 16, E).transpose(0, 2, 1).reshape(NW, E * 16)

    xs_pad, pos3 = _sc_scatter_x(x, ids32, rank, bw_all, B_pad, NW)
    out_s = _grouped_matmul(off, xs_pad, W, b)
    out = _sc_gather_out(out_s, pos3, B, NW)
    return out


# speedup vs baseline: 1.1131x; 1.0048x over previous
import functools

import jax
import jax.numpy as jnp
from jax import lax
from jax.experimental import pallas as pl
from jax.experimental.pallas import tpu as pltpu
from jax.experimental.pallas import tpu_sc as plsc

_C = 16
_TT = 256
_KC = 128


def _tc_body(off_ref, xs_ref, w_ref, b_ref, out_ref, acc_ref):
    g = pl.program_id(0)
    ncols = out_ref.shape[1]

    @pl.when(g == 0)
    def _init():
        out_ref[...] = jnp.zeros(out_ref.shape, jnp.float32)

    rs = off_ref[g * _C]
    re = off_ref[g * _C + _C]
    base0 = (rs // 8) * 8
    nblk = (re - base0 + _TT - 1) // _TT

    def blk(k, carry):
        base = base0 + k * _TT
        xblk = xs_ref[pl.ds(base, _TT), :]
        riota = base + lax.broadcasted_iota(jnp.int32, (_TT, 1), 0)
        acc = jnp.zeros((_TT, ncols), jnp.float32)
        for j in range(_C):
            oe = off_ref[g * _C + j]
            oe1 = off_ref[g * _C + j + 1]
            m = (riota >= oe) & (riota < oe1)
            dotj = jnp.dot(xblk, w_ref[j], precision=lax.Precision.DEFAULT, preferred_element_type=jnp.float32)
            brow = b_ref[pl.ds(j, 1), :]
            acc = acc + jnp.where(m, dotj + brow, 0.0)
        out_ref[pl.ds(base, _TT), :] = out_ref[pl.ds(base, _TT), :] + acc
        return carry

    lax.fori_loop(0, nblk, blk, 0)


def _grouped_matmul(off, xs_pad, W, b, interpret=False):
    E, D, _ = W.shape
    B_pad = xs_pad.shape[0]
    grid = (E // _C,)
    grid_spec = pltpu.PrefetchScalarGridSpec(
        num_scalar_prefetch=1,
        grid=grid,
        in_specs=[
            pl.BlockSpec((B_pad, D), lambda g, off_ref: (0, 0)),
            pl.BlockSpec((_C, D, D), lambda g, off_ref: (g, 0, 0)),
            pl.BlockSpec((_C, D), lambda g, off_ref: (g, 0)),
        ],
        out_specs=pl.BlockSpec((B_pad, D), lambda g, off_ref: (0, 0)),
        scratch_shapes=[pltpu.VMEM((_TT, D), jnp.float32)],
    )
    return pl.pallas_call(
        _tc_body,
        grid_spec=grid_spec,
        out_shape=jax.ShapeDtypeStruct((B_pad, D), jnp.float32),
        interpret=interpret,
    )(off, xs_pad, W, b)


def _sc_mesh():
    return plsc.VectorSubcoreMesh(core_axis_name="c", subcore_axis_name="s")


def _sc_hist_rank(ids, E, NW):
    B = ids.shape[0]
    tpw = B // NW
    TPL = tpw // 16

    @functools.partial(
        pl.kernel,
        mesh=_sc_mesh(),
        out_type=(
            jax.ShapeDtypeStruct((NW, E * 16), jnp.int32),
            jax.ShapeDtypeStruct((B,), jnp.int32),
        ),
        scratch_types=[
            pltpu.VMEM((tpw,), jnp.int32),
            pltpu.VMEM((E * 16,), jnp.int32),
            pltpu.VMEM((tpw,), jnp.int32),
        ],
        compiler_params=pltpu.CompilerParams(needs_layout_passes=False),
    )
    def k(ids_hbm, hist_hbm, rank_hbm, ids_v, hist_v, rank_v):
        wid = lax.axis_index("s") * 2 + lax.axis_index("c")
        base = wid * tpw
        lane = lax.iota(jnp.int32, 16)
        pltpu.sync_copy(ids_hbm.at[pl.ds(base, tpw)], ids_v)

        def zero(i, _):
            for u in range(16):
                hist_v[pl.ds(i * 256 + u * 16, 16)] = jnp.zeros((16,), jnp.int32)
            return 0

        lax.fori_loop(0, E // 16, zero, 0)

        def body(j, _):
            idx = lane * TPL + j
            idvec = plsc.load_gather(ids_v, [idx])
            hidx = idvec * 16 + lane
            c = plsc.load_gather(hist_v, [hidx])
            plsc.store_scatter(hist_v, [hidx], c + 1)
            plsc.store_scatter(rank_v, [idx], c)
            return 0

        lax.fori_loop(0, TPL, body, 0)
        pltpu.sync_copy(hist_v, hist_hbm.at[wid])
        pltpu.sync_copy(rank_v, rank_hbm.at[pl.ds(base, tpw)])

    return k(ids)


def _sc_scatter_x(x, ids, rank, bw_all, B_pad, NW):
    B, D = x.shape
    tpw = B // NW
    TPL = tpw // 16
    nchunk = tpw // _KC
    E16 = bw_all.shape[1]

    @functools.partial(
        pl.kernel,
        mesh=_sc_mesh(),
        out_type=(
            jax.ShapeDtypeStruct((B_pad, D), jnp.float32),
            jax.ShapeDtypeStruct((NW, nchunk, _KC), jnp.int32),
        ),
        scratch_types=[
            pltpu.VMEM((tpw,), jnp.int32),
            pltpu.VMEM((tpw,), jnp.int32),
            pltpu.VMEM((E16,), jnp.int32),
            pltpu.VMEM((nchunk, _KC), jnp.int32),
            pltpu.VMEM((tpw, D), jnp.float32),
            pltpu.SemaphoreType.DMA,
        ],
        compiler_params=pltpu.CompilerParams(needs_layout_passes=False),
    )
    def k(x_hbm, ids_hbm, rank_hbm, bw_hbm, xs_hbm, pos_hbm,
          ids_v, rank_v, bw_v, pos_v, rows_v, sem):
        wid = lax.axis_index("s") * 2 + lax.axis_index("c")
        base = wid * tpw
        pltpu.sync_copy(ids_hbm.at[pl.ds(base, tpw)], ids_v)
        pltpu.sync_copy(rank_hbm.at[pl.ds(base, tpw)], rank_v)
        pltpu.sync_copy(bw_hbm.at[wid], bw_v)
        pltpu.sync_copy(x_hbm.at[pl.ds(base, tpw)], rows_v)
        for g in range(tpw // 16):
            sublane = (g * 16) // TPL
            sl = pl.ds(g * 16, 16)
            idvec = ids_v[sl]
            rvec = rank_v[sl]
            bvec = plsc.load_gather(bw_v, [idvec * 16 + sublane])
            pos_v[g // (_KC // 16), pl.ds((g % (_KC // 16)) * 16, 16)] = (
                bvec + rvec
            )
        copies = [
            pltpu.async_copy(
                rows_v.at[pl.ds(j * _KC, _KC)],
                xs_hbm.at[pos_v.at[j]],
                sem,
            )
            for j in range(nchunk)
        ]
        for c in copies:
            c.wait()
        pltpu.sync_copy(pos_v, pos_hbm.at[wid])

    return k(x, ids, rank, bw_all)


def _sc_gather_out(out_s, pos3, B, NW):
    D = out_s.shape[1]
    nchunk = pos3.shape[1]
    tpw = nchunk * _KC

    @functools.partial(
        pl.kernel,
        mesh=_sc_mesh(),
        out_type=jax.ShapeDtypeStruct((B, D), jnp.float32),
        scratch_types=[
            pltpu.VMEM((nchunk, _KC), jnp.int32),
            pltpu.VMEM((tpw, D), jnp.float32),
            pltpu.SemaphoreType.DMA,
        ],
    )
    def k(src_hbm, pos_hbm, out_hbm, idx_v, rows_v, sem):
        wid = lax.axis_index("s") * 2 + lax.axis_index("c")
        base = wid * tpw
        pltpu.sync_copy(pos_hbm.at[wid], idx_v)
        copies = [
            pltpu.async_copy(
                src_hbm.at[idx_v.at[j]],
                rows_v.at[pl.ds(j * _KC, _KC)],
                sem,
            )
            for j in range(nchunk)
        ]
        for c in copies:
            c.wait()
        pltpu.sync_copy(rows_v, out_hbm.at[pl.ds(base, tpw)])

    return k(out_s, pos3)


def kernel(x, ids, W, b, interpret=False):
    B, D = x.shape
    E = W.shape[0]
    B_pad = B + _TT
    NW = 32
    ids32 = ids.astype(jnp.int32)

    if interpret:
        tok = lax.iota(jnp.int32, B)
        sorted_ids, order = lax.sort((ids32, tok), num_keys=1)
        off = jnp.searchsorted(
            sorted_ids, lax.iota(jnp.int32, E + 1), side="left"
        ).astype(jnp.int32)
        xs_pad = jnp.pad(jnp.take(x, order, axis=0), ((0, _TT), (0, 0)))
        out_s = _grouped_matmul(off, xs_pad, W, b, interpret=True)
        return jnp.zeros((B, D), jnp.float32).at[order].set(out_s[:B])

    hist, rank = _sc_hist_rank(ids32, E, NW)
    hsw = hist.reshape(NW, E, 16).transpose(0, 2, 1).reshape(NW * 16, E)
    colsum = hsw.sum(axis=0)
    off = jnp.concatenate(
        [jnp.zeros((1,), jnp.int32), jnp.cumsum(colsum, dtype=jnp.int32)]
    )
    basew = (
        off[:E][None, :]
        + jnp.cumsum(hsw, axis=0, dtype=jnp.int32)
        - hsw
    )
    bw_all = basew.reshape(NW, 16, E).transpose(0, 2, 1).reshape(NW, E * 16)

    xs_pad, pos3 = _sc_scatter_x(x, ids32, rank, bw_all, B_pad, NW)
    out_s = _grouped_matmul(off, xs_pad, W, b)
    out = _sc_gather_out(out_s, pos3, B, NW)
    return out


# speedup vs baseline: 1.2893x; 1.1583x over previous
import functools

import jax
import jax.numpy as jnp
from jax import lax
from jax.experimental import pallas as pl
from jax.experimental.pallas import tpu as pltpu
from jax.experimental.pallas import tpu_sc as plsc

_C = 16
_TT = 256
_KC = 128


def _tc_body(off_ref, xs_ref, w_ref, b_ref, out_ref, acc_ref):
    g = pl.program_id(0)
    ncols = out_ref.shape[1]

    @pl.when(g == 0)
    def _init():
        out_ref[...] = jnp.zeros(out_ref.shape, jnp.float32)

    rs = off_ref[g * _C]
    re = off_ref[g * _C + _C]
    base0 = (rs // 8) * 8
    nblk = (re - base0 + _TT - 1) // _TT

    def blk(k, carry):
        base = base0 + k * _TT
        xblk = xs_ref[pl.ds(base, _TT), :]
        riota = base + lax.broadcasted_iota(jnp.int32, (_TT, 1), 0)
        acc = jnp.zeros((_TT, ncols), jnp.float32)
        for j in range(_C):
            oe = off_ref[g * _C + j]
            oe1 = off_ref[g * _C + j + 1]
            m = (riota >= oe) & (riota < oe1)
            dotj = jnp.dot(xblk, w_ref[j], preferred_element_type=jnp.float32)
            brow = b_ref[pl.ds(j, 1), :]
            acc = acc + jnp.where(m, dotj + brow, 0.0)
        out_ref[pl.ds(base, _TT), :] = out_ref[pl.ds(base, _TT), :] + acc
        return carry

    lax.fori_loop(0, nblk, blk, 0)


def _grouped_matmul(off, xs_pad, W, b, interpret=False):
    E, D, _ = W.shape
    B_pad = xs_pad.shape[0]
    grid = (E // _C,)
    grid_spec = pltpu.PrefetchScalarGridSpec(
        num_scalar_prefetch=1,
        grid=grid,
        in_specs=[
            pl.BlockSpec((B_pad, D), lambda g, off_ref: (0, 0)),
            pl.BlockSpec((_C, D, D), lambda g, off_ref: (g, 0, 0)),
            pl.BlockSpec((_C, D), lambda g, off_ref: (g, 0)),
        ],
        out_specs=pl.BlockSpec((B_pad, D), lambda g, off_ref: (0, 0)),
        scratch_shapes=[pltpu.VMEM((_TT, D), jnp.float32)],
    )
    return pl.pallas_call(
        _tc_body,
        grid_spec=grid_spec,
        out_shape=jax.ShapeDtypeStruct((B_pad, D), jnp.float32),
        interpret=interpret,
    )(off, xs_pad, W, b)


def _sc_mesh():
    return plsc.VectorSubcoreMesh(core_axis_name="c", subcore_axis_name="s")


def _sc_hist_rank(ids, E, NW):
    B = ids.shape[0]
    tpw = B // NW
    TPL = tpw // 16

    @functools.partial(
        pl.kernel,
        mesh=_sc_mesh(),
        out_type=(
            jax.ShapeDtypeStruct((NW, E * 16), jnp.int32),
            jax.ShapeDtypeStruct((B,), jnp.int32),
        ),
        scratch_types=[
            pltpu.VMEM((tpw,), jnp.int32),
            pltpu.VMEM((E * 16,), jnp.int32),
            pltpu.VMEM((tpw,), jnp.int32),
        ],
        compiler_params=pltpu.CompilerParams(needs_layout_passes=False),
    )
    def k(ids_hbm, hist_hbm, rank_hbm, ids_v, hist_v, rank_v):
        wid = lax.axis_index("s") * 2 + lax.axis_index("c")
        base = wid * tpw
        lane = lax.iota(jnp.int32, 16)
        pltpu.sync_copy(ids_hbm.at[pl.ds(base, tpw)], ids_v)

        def zero(i, _):
            for u in range(16):
                hist_v[pl.ds(i * 256 + u * 16, 16)] = jnp.zeros((16,), jnp.int32)
            return 0

        lax.fori_loop(0, E // 16, zero, 0)

        def body(j, _):
            idx = lane * TPL + j
            idvec = plsc.load_gather(ids_v, [idx])
            hidx = idvec + lane * E
            c = plsc.load_gather(hist_v, [hidx])
            plsc.store_scatter(hist_v, [hidx], c + 1)
            plsc.store_scatter(rank_v, [idx], c)
            return 0

        lax.fori_loop(0, TPL, body, 0)
        pltpu.sync_copy(hist_v, hist_hbm.at[wid])
        pltpu.sync_copy(rank_v, rank_hbm.at[pl.ds(base, tpw)])

    return k(ids)


def _sc_scatter_x(x, ids, rank, bw_all, B_pad, NW):
    B, D = x.shape
    tpw = B // NW
    TPL = tpw // 16
    nchunk = tpw // _KC
    E16 = bw_all.shape[1]
    E = E16 // 16

    @functools.partial(
        pl.kernel,
        mesh=_sc_mesh(),
        out_type=(
            jax.ShapeDtypeStruct((B_pad, D), jnp.float32),
            jax.ShapeDtypeStruct((NW, nchunk, _KC), jnp.int32),
        ),
        scratch_types=[
            pltpu.VMEM((tpw,), jnp.int32),
            pltpu.VMEM((tpw,), jnp.int32),
            pltpu.VMEM((E16,), jnp.int32),
            pltpu.VMEM((nchunk, _KC), jnp.int32),
            pltpu.VMEM((tpw, D), jnp.float32),
            pltpu.SemaphoreType.DMA,
        ],
        compiler_params=pltpu.CompilerParams(needs_layout_passes=False),
    )
    def k(x_hbm, ids_hbm, rank_hbm, bw_hbm, xs_hbm, pos_hbm,
          ids_v, rank_v, bw_v, pos_v, rows_v, sem):
        wid = lax.axis_index("s") * 2 + lax.axis_index("c")
        base = wid * tpw
        pltpu.sync_copy(ids_hbm.at[pl.ds(base, tpw)], ids_v)
        pltpu.sync_copy(rank_hbm.at[pl.ds(base, tpw)], rank_v)
        pltpu.sync_copy(bw_hbm.at[wid], bw_v)
        pltpu.sync_copy(x_hbm.at[pl.ds(base, tpw)], rows_v)
        for g in range(tpw // 16):
            sublane = (g * 16) // TPL
            sl = pl.ds(g * 16, 16)
            idvec = ids_v[sl]
            rvec = rank_v[sl]
            bvec = plsc.load_gather(bw_v, [idvec + sublane * E])
            pos_v[g // (_KC // 16), pl.ds((g % (_KC // 16)) * 16, 16)] = (
                bvec + rvec
            )
        copies = [
            pltpu.async_copy(
                rows_v.at[pl.ds(j * _KC, _KC)],
                xs_hbm.at[pos_v.at[j]],
                sem,
            )
            for j in range(nchunk)
        ]
        for c in copies:
            c.wait()
        pltpu.sync_copy(pos_v, pos_hbm.at[wid])

    return k(x, ids, rank, bw_all)


def _sc_gather_out(out_s, pos3, B, NW):
    D = out_s.shape[1]
    nchunk = pos3.shape[1]
    tpw = nchunk * _KC

    @functools.partial(
        pl.kernel,
        mesh=_sc_mesh(),
        out_type=jax.ShapeDtypeStruct((B, D), jnp.float32),
        scratch_types=[
            pltpu.VMEM((nchunk, _KC), jnp.int32),
            pltpu.VMEM((tpw, D), jnp.float32),
            pltpu.SemaphoreType.DMA,
        ],
    )
    def k(src_hbm, pos_hbm, out_hbm, idx_v, rows_v, sem):
        wid = lax.axis_index("s") * 2 + lax.axis_index("c")
        base = wid * tpw
        pltpu.sync_copy(pos_hbm.at[wid], idx_v)
        copies = [
            pltpu.async_copy(
                src_hbm.at[idx_v.at[j]],
                rows_v.at[pl.ds(j * _KC, _KC)],
                sem,
            )
            for j in range(nchunk)
        ]
        for c in copies:
            c.wait()
        pltpu.sync_copy(rows_v, out_hbm.at[pl.ds(base, tpw)])

    return k(out_s, pos3)


def kernel(x, ids, W, b, interpret=False):
    B, D = x.shape
    E = W.shape[0]
    B_pad = B + _TT
    NW = 32
    ids32 = ids.astype(jnp.int32)

    if interpret:
        tok = lax.iota(jnp.int32, B)
        sorted_ids, order = lax.sort((ids32, tok), num_keys=1)
        off = jnp.searchsorted(
            sorted_ids, lax.iota(jnp.int32, E + 1), side="left"
        ).astype(jnp.int32)
        xs_pad = jnp.pad(jnp.take(x, order, axis=0), ((0, _TT), (0, 0)))
        out_s = _grouped_matmul(off, xs_pad, W, b, interpret=True)
        return jnp.zeros((B, D), jnp.float32).at[order].set(out_s[:B])

    hist, rank = _sc_hist_rank(ids32, E, NW)
    hsw = hist.reshape(NW * 16, E)
    colsum = hsw.sum(axis=0)
    off = jnp.concatenate(
        [jnp.zeros((1,), jnp.int32), jnp.cumsum(colsum, dtype=jnp.int32)]
    )
    basew = (
        off[:E][None, :]
        + jnp.cumsum(hsw, axis=0, dtype=jnp.int32)
        - hsw
    )
    bw_all = basew.reshape(NW, 16 * E)

    xs_pad, pos3 = _sc_scatter_x(x, ids32, rank, bw_all, B_pad, NW)
    out_s = _grouped_matmul(off, xs_pad, W, b)
    out = _sc_gather_out(out_s, pos3, B, NW)
    return out


# speedup vs baseline: 1.2930x; 1.0029x over previous
import functools

import jax
import jax.numpy as jnp
from jax import lax
from jax.experimental import pallas as pl
from jax.experimental.pallas import tpu as pltpu
from jax.experimental.pallas import tpu_sc as plsc

_C = 16
_TT = 256
_KC = 128


def _tc_body(off_ref, xs_ref, w_ref, b_ref, out_ref, acc_ref):
    g = pl.program_id(0)
    ncols = out_ref.shape[1]

    @pl.when(g == 0)
    def _init():
        out_ref[...] = jnp.zeros(out_ref.shape, jnp.float32)

    rs = off_ref[g * _C]
    re = off_ref[g * _C + _C]
    base0 = (rs // 8) * 8
    nblk = (re - base0 + _TT - 1) // _TT

    def blk(k, carry):
        base = base0 + k * _TT
        xblk = xs_ref[pl.ds(base, _TT), :]
        riota = base + lax.broadcasted_iota(jnp.int32, (_TT, 1), 0)
        acc = jnp.zeros((_TT, ncols), jnp.float32)
        for j in range(_C):
            oe = off_ref[g * _C + j]
            oe1 = off_ref[g * _C + j + 1]
            m = (riota >= oe) & (riota < oe1)
            dotj = jnp.dot(xblk, w_ref[j], preferred_element_type=jnp.float32)
            brow = b_ref[pl.ds(j, 1), :]
            acc = acc + jnp.where(m, dotj + brow, 0.0)
        out_ref[pl.ds(base, _TT), :] = out_ref[pl.ds(base, _TT), :] + acc
        return carry

    lax.fori_loop(0, nblk, blk, 0)


def _grouped_matmul(off, xs_pad, W, b):
    E, D, _ = W.shape
    B_pad = xs_pad.shape[0]
    grid = (E // _C,)
    grid_spec = pltpu.PrefetchScalarGridSpec(
        num_scalar_prefetch=1,
        grid=grid,
        in_specs=[
            pl.BlockSpec((B_pad, D), lambda g, off_ref: (0, 0)),
            pl.BlockSpec((_C, D, D), lambda g, off_ref: (g, 0, 0)),
            pl.BlockSpec((_C, D), lambda g, off_ref: (g, 0)),
        ],
        out_specs=pl.BlockSpec((B_pad, D), lambda g, off_ref: (0, 0)),
        scratch_shapes=[pltpu.VMEM((_TT, D), jnp.float32)],
    )
    return pl.pallas_call(
        _tc_body,
        grid_spec=grid_spec,
        out_shape=jax.ShapeDtypeStruct((B_pad, D), jnp.float32),
    )(off, xs_pad, W, b)


def _sc_mesh():
    return plsc.VectorSubcoreMesh(core_axis_name="c", subcore_axis_name="s")


def _sc_hist_rank(ids, E, NW):
    B = ids.shape[0]
    tpw = B // NW
    TPL = tpw // 16

    @functools.partial(
        pl.kernel,
        mesh=_sc_mesh(),
        out_type=(
            jax.ShapeDtypeStruct((NW, E * 16), jnp.int32),
            jax.ShapeDtypeStruct((B,), jnp.int32),
        ),
        scratch_types=[
            pltpu.VMEM((tpw,), jnp.int32),
            pltpu.VMEM((E * 16,), jnp.int32),
            pltpu.VMEM((tpw,), jnp.int32),
        ],
        compiler_params=pltpu.CompilerParams(needs_layout_passes=False),
    )
    def k(ids_hbm, hist_hbm, rank_hbm, ids_v, hist_v, rank_v):
        wid = lax.axis_index("s") * 2 + lax.axis_index("c")
        base = wid * tpw
        lane = lax.iota(jnp.int32, 16)
        pltpu.sync_copy(ids_hbm.at[pl.ds(base, tpw)], ids_v)

        def zero(i, _):
            for u in range(16):
                hist_v[pl.ds(i * 256 + u * 16, 16)] = jnp.zeros((16,), jnp.int32)
            return 0

        lax.fori_loop(0, E // 16, zero, 0)

        def body(j, _):
            idx = lane * TPL + j
            idvec = plsc.load_gather(ids_v, [idx])
            hidx = idvec + lane * E
            c = plsc.load_gather(hist_v, [hidx])
            plsc.store_scatter(hist_v, [hidx], c + 1)
            plsc.store_scatter(rank_v, [idx], c)
            return 0

        lax.fori_loop(0, TPL, body, 0)
        pltpu.sync_copy(hist_v, hist_hbm.at[wid])
        pltpu.sync_copy(rank_v, rank_hbm.at[pl.ds(base, tpw)])

    return k(ids)


def _sc_scatter_x(x, ids, rank, bw_all, B_pad, NW):
    B, D = x.shape
    tpw = B // NW
    TPL = tpw // 16
    nchunk = tpw // _KC
    E16 = bw_all.shape[1]
    E = E16 // 16

    @functools.partial(
        pl.kernel,
        mesh=_sc_mesh(),
        out_type=(
            jax.ShapeDtypeStruct((B_pad, D), jnp.float32),
            jax.ShapeDtypeStruct((NW, nchunk, _KC), jnp.int32),
        ),
        scratch_types=[
            pltpu.VMEM((tpw,), jnp.int32),
            pltpu.VMEM((tpw,), jnp.int32),
            pltpu.VMEM((E16,), jnp.int32),
            pltpu.VMEM((nchunk, _KC), jnp.int32),
            pltpu.VMEM((tpw, D), jnp.float32),
            pltpu.SemaphoreType.DMA,
        ],
        compiler_params=pltpu.CompilerParams(needs_layout_passes=False),
    )
    def k(x_hbm, ids_hbm, rank_hbm, bw_hbm, xs_hbm, pos_hbm,
          ids_v, rank_v, bw_v, pos_v, rows_v, sem):
        wid = lax.axis_index("s") * 2 + lax.axis_index("c")
        base = wid * tpw
        pltpu.sync_copy(ids_hbm.at[pl.ds(base, tpw)], ids_v)
        pltpu.sync_copy(rank_hbm.at[pl.ds(base, tpw)], rank_v)
        pltpu.sync_copy(bw_hbm.at[wid], bw_v)
        pltpu.sync_copy(x_hbm.at[pl.ds(base, tpw)], rows_v)
        for g in range(tpw // 16):
            sublane = (g * 16) // TPL
            sl = pl.ds(g * 16, 16)
            idvec = ids_v[sl]
            rvec = rank_v[sl]
            bvec = plsc.load_gather(bw_v, [idvec + sublane * E])
            pos_v[g // (_KC // 16), pl.ds((g % (_KC // 16)) * 16, 16)] = (
                bvec + rvec
            )
        copies = [
            pltpu.async_copy(
                rows_v.at[pl.ds(j * _KC, _KC)],
                xs_hbm.at[pos_v.at[j]],
                sem,
            )
            for j in range(nchunk)
        ]
        for c in copies:
            c.wait()
        pltpu.sync_copy(pos_v, pos_hbm.at[wid])

    return k(x, ids, rank, bw_all)


def _sc_gather_out(out_s, pos3, B, NW):
    D = out_s.shape[1]
    nchunk = pos3.shape[1]
    tpw = nchunk * _KC

    @functools.partial(
        pl.kernel,
        mesh=_sc_mesh(),
        out_type=jax.ShapeDtypeStruct((B, D), jnp.float32),
        scratch_types=[
            pltpu.VMEM((nchunk, _KC), jnp.int32),
            pltpu.VMEM((tpw, D), jnp.float32),
            pltpu.SemaphoreType.DMA,
        ],
    )
    def k(src_hbm, pos_hbm, out_hbm, idx_v, rows_v, sem):
        wid = lax.axis_index("s") * 2 + lax.axis_index("c")
        base = wid * tpw
        pltpu.sync_copy(pos_hbm.at[wid], idx_v)
        copies = [
            pltpu.async_copy(
                src_hbm.at[idx_v.at[j]],
                rows_v.at[pl.ds(j * _KC, _KC)],
                sem,
            )
            for j in range(nchunk)
        ]
        for c in copies:
            c.wait()
        pltpu.sync_copy(rows_v, out_hbm.at[pl.ds(base, tpw)])

    return k(out_s, pos3)


def kernel(x, ids, W, b):
    B, D = x.shape
    E = W.shape[0]
    B_pad = B + _TT
    NW = 32
    ids32 = ids.astype(jnp.int32)

    hist, rank = _sc_hist_rank(ids32, E, NW)
    hsw = hist.reshape(NW * 16, E)
    colsum = hsw.sum(axis=0)
    off = jnp.concatenate(
        [jnp.zeros((1,), jnp.int32), jnp.cumsum(colsum, dtype=jnp.int32)]
    )
    basew = (
        off[:E][None, :]
        + jnp.cumsum(hsw, axis=0, dtype=jnp.int32)
        - hsw
    )
    bw_all = basew.reshape(NW, 16 * E)

    xs_pad, pos3 = _sc_scatter_x(x, ids32, rank, bw_all, B_pad, NW)
    out_s = _grouped_matmul(off, xs_pad, W, b)
    out = _sc_gather_out(out_s, pos3, B, NW)
    return out
